# baseline (speedup 1.0000x reference)
import functools

import jax
import jax.numpy as jnp
from jax import lax
from jax.experimental import pallas as pl
from jax.experimental.pallas import tpu as pltpu
from jax.experimental.pallas import tpu_sc as plsc

_D = 128
_B = 64
_K = 16384
_N = 100000
_T = 0.07
_M = 0.5
_RB = 4096
_CHUNK = 4096
_F32 = jnp.float32


def _embed_body(fs_ref, ft_ref, ws_ref, bs_ref, wt_ref, bt_ref, v1_ref, v2_ref):
    dn = (((1,), (1,)), ((), ()))
    x1 = lax.dot_general(fs_ref[...], ws_ref[...], dn,
                         preferred_element_type=_F32) + bs_ref[...]
    n1 = jnp.power(jnp.sum(x1 * x1, axis=1, keepdims=True), 0.5)
    v1_ref[...] = x1 / n1
    x2 = lax.dot_general(ft_ref[...], wt_ref[...], dn,
                         preferred_element_type=_F32) + bt_ref[...]
    n2 = jnp.power(jnp.sum(x2 * x2, axis=1, keepdims=True), 0.5)
    v2_ref[...] = x2 / n2


def _embed(f_s, f_t, W_s, b_s, W_t, b_t):
    return pl.pallas_call(
        _embed_body,
        out_shape=(jax.ShapeDtypeStruct((_B, _D), _F32),
                   jax.ShapeDtypeStruct((_B, _D), _F32)),
    )(f_s, f_t, W_s, b_s.reshape(1, _D), W_t, b_t.reshape(1, _D))


def _rowgather(idx, memory_v1, memory_v2):
    mesh = plsc.VectorSubcoreMesh(core_axis_name="c", subcore_axis_name="s")

    @functools.partial(
        pl.kernel,
        mesh=mesh,
        out_type=(jax.ShapeDtypeStruct((_B, _D), _F32),
                  jax.ShapeDtypeStruct((_B, _D), _F32)),
        scratch_types=[pltpu.VMEM((8,), jnp.int32),
                       pltpu.VMEM((8, _D), _F32),
                       pltpu.SemaphoreType.DMA],
    )
    def k(idx_hbm, mem1_hbm, mem2_hbm, m1_hbm, m2_hbm, idx_v, rows_v, sem):
        w = lax.axis_index("s") * 2 + lax.axis_index("c")

        @pl.when(w < 8)
        def _():
            pltpu.sync_copy(idx_hbm.at[pl.ds(w * 8, 8)], idx_v)
            pltpu.async_copy(mem1_hbm.at[idx_v], rows_v, sem).wait()
            pltpu.sync_copy(rows_v, m1_hbm.at[pl.ds(w * 8, 8)])
            pltpu.async_copy(mem2_hbm.at[idx_v], rows_v, sem).wait()
            pltpu.sync_copy(rows_v, m2_hbm.at[pl.ds(w * 8, 8)])

    return k(idx, memory_v1, memory_v2)


def _prep_body(idx_ref, m1_ref, m2_ref, v1_ref, v2_ref,
               l1_ref, l2_ref, pos1_ref, pos2_ref):
    ii = idx_ref[...]
    eq = (ii.T == ii)
    row = lax.broadcasted_iota(jnp.int32, (_B, _B), 0)
    col = lax.broadcasted_iota(jnp.int32, (_B, _B), 1)
    dup_after = jnp.any(eq & (col > row), axis=1, keepdims=True)
    win = (eq & jnp.logical_not(dup_after.T)).astype(_F32)

    m1 = m1_ref[...]
    m2 = m2_ref[...]
    v1 = v1_ref[...]
    v2 = v2_ref[...]
    l1 = m1 * _M + v1 * (1.0 - _M)
    l1 = l1 / jnp.power(jnp.sum(l1 * l1, axis=1, keepdims=True), 0.5)
    l2 = m2 * _M + v2 * (1.0 - _M)
    l2 = l2 / jnp.power(jnp.sum(l2 * l2, axis=1, keepdims=True), 0.5)
    l1_ref[...] = jnp.dot(win, l1, preferred_element_type=_F32)
    l2_ref[...] = jnp.dot(win, l2, preferred_element_type=_F32)
    pos1_ref[...] = jnp.broadcast_to(
        jnp.sum(m2 * v1, axis=1, keepdims=True), (_B, _D))
    pos2_ref[...] = jnp.broadcast_to(
        jnp.sum(m1 * v2, axis=1, keepdims=True), (_B, _D))


def _prep(idx, m1, m2, v1, v2):
    return pl.pallas_call(
        _prep_body,
        out_shape=tuple(jax.ShapeDtypeStruct((_B, _D), _F32) for _ in range(4)),
    )(idx.reshape(1, _B), m1, m2, v1, v2)


def _mm_bank_body(idx_ref, v_ref, l_ref, m_ref, p_ref, o_ref):
    dn = (((1,), (1,)), ((), ()))
    m = m_ref[...]
    p_ref[...] = lax.dot_general(v_ref[...], m, dn,
                                 preferred_element_type=_F32)
    o_ref[...] = m
    base = pl.program_id(0) * _RB
    for b in range(_B):
        j = idx_ref[b] - base

        @pl.when((j >= 0) & (j < _RB))
        def _():
            o_ref[pl.ds(j, 1), :] = l_ref[pl.ds(b, 1), :]


def _mm_p_body(v_ref, m_ref, p_ref):
    dn = (((1,), (1,)), ((), ()))
    p_ref[...] = lax.dot_general(v_ref[...], m_ref[...], dn,
                                 preferred_element_type=_F32)


def _mm_p(v, memory):
    grid = (pl.cdiv(_N, _RB),)
    full = pl.BlockSpec((_B, _D), lambda r: (0, 0))
    memb = pl.BlockSpec((_RB, _D), lambda r: (r, 0))
    pblk = pl.BlockSpec((_B, _RB), lambda r: (0, r))
    return pl.pallas_call(
        _mm_p_body,
        grid=grid,
        in_specs=[full, memb],
        out_specs=pblk,
        out_shape=jax.ShapeDtypeStruct((_B, _N), _F32),
        compiler_params=pltpu.CompilerParams(
            dimension_semantics=("arbitrary",)),
    )(v, memory)


def _copy_one_body(idx_ref, l_ref, m_ref, dep_ref, o_ref):
    o_ref[...] = m_ref[...]
    base = pl.program_id(0) * _RB
    for b in range(_B):
        j = idx_ref[b] - base

        @pl.when((j >= 0) & (j < _RB))
        def _():
            o_ref[pl.ds(j, 1), :] = l_ref[pl.ds(b, 1), :]


def _copy_scatter_one(idx, l, memory, dep):
    grid = (pl.cdiv(_N, _RB),)
    full = pl.BlockSpec((_B, _D), lambda r: (0, 0))
    memb = pl.BlockSpec((_RB, _D), lambda r: (r, 0))
    return pl.pallas_call(
        _copy_one_body,
        grid=grid,
        in_specs=[pl.BlockSpec(memory_space=pltpu.SMEM), full, memb,
                  pl.BlockSpec((8, 128), lambda r: (0, 0))],
        out_specs=memb,
        out_shape=jax.ShapeDtypeStruct((_N, _D), _F32),
        compiler_params=pltpu.CompilerParams(
            dimension_semantics=("arbitrary",)),
    )(idx, l, memory, dep)


def _mm_bank(idx, v, l, memory):
    grid = (pl.cdiv(_N, _RB),)
    full = pl.BlockSpec((_B, _D), lambda r: (0, 0))
    memb = pl.BlockSpec((_RB, _D), lambda r: (r, 0))
    pblk = pl.BlockSpec((_B, _RB), lambda r: (0, r))
    return pl.pallas_call(
        _mm_bank_body,
        grid=grid,
        in_specs=[pl.BlockSpec(memory_space=pltpu.SMEM), full, full, memb],
        out_specs=[pblk, memb],
        out_shape=(jax.ShapeDtypeStruct((_B, _N), _F32),
                   jax.ShapeDtypeStruct((_N, _D), _F32)),
        compiler_params=pltpu.CompilerParams(
            dimension_semantics=("arbitrary",)),
    )(idx, v, l, memory)


def _gather_bank(p, cidx):
    mesh = plsc.VectorSubcoreMesh(core_axis_name="c", subcore_axis_name="s")

    @functools.partial(
        pl.kernel,
        mesh=mesh,
        out_type=jax.ShapeDtypeStruct((_B, _K), _F32),
        scratch_types=[pltpu.VMEM((_N,), _F32),
                       pltpu.VMEM((2, _CHUNK), jnp.int32),
                       pltpu.VMEM((2, _CHUNK), _F32),
                       pltpu.SemaphoreType.DMA,
                       pltpu.SemaphoreType.DMA],
        compiler_params=pltpu.CompilerParams(needs_layout_passes=False),
    )
    def k(p_hbm, cidx_hbm, g_hbm,
          p_buf, idx_buf, out_buf, sem_i, sem_o):
        w = lax.axis_index("s") * 2 + lax.axis_index("c")

        nch = _K // _CHUNK

        def gather_row(b):
            icp = [pltpu.async_copy(cidx_hbm.at[b, pl.ds(0, _CHUNK)],
                                    idx_buf.at[0], sem_i)]
            pltpu.sync_copy(p_hbm.at[b], p_buf)
            ocp = [None] * nch
            for c in range(nch):
                s = c % 2
                icp[c].wait()
                if c + 1 < nch:
                    icp.append(pltpu.async_copy(
                        cidx_hbm.at[b, pl.ds((c + 1) * _CHUNK, _CHUNK)],
                        idx_buf.at[(c + 1) % 2], sem_i))
                if c >= 2:
                    ocp[c - 2].wait()

                def gl(j, carry):
                    for u in range(4):
                        o = j * 64 + u * 16
                        vi = idx_buf[s, pl.ds(o, 16)]
                        out_buf[s, pl.ds(o, 16)] = \
                            plsc.load_gather(p_buf, [vi])
                    return carry

                lax.fori_loop(0, _CHUNK // 64, gl, 0)
                ocp[c] = pltpu.async_copy(
                    out_buf.at[s], g_hbm.at[b, pl.ds(c * _CHUNK, _CHUNK)],
                    sem_o)
            ocp[nch - 2].wait()
            ocp[nch - 1].wait()

        gather_row(w * 2)
        gather_row(w * 2 + 1)

    return k(p, cidx)


def _loss_body(g1_ref, g2_ref, pos1_ref, pos2_ref, loss_ref):
    inv_t = 1.0 / _T
    scale = float(_N) / float(_B * (_K + 1))

    def ce(neg_raw, pos_raw):
        en = jnp.exp(neg_raw * inv_t)
        ep = jnp.exp(pos_raw * inv_t)
        z = jnp.sum(jnp.sum(en, axis=1, keepdims=True) + ep) * scale
        on = en / z
        op = ep / z
        mx = jnp.maximum(jnp.max(on, axis=1, keepdims=True), op)
        logz = jnp.log(jnp.sum(jnp.exp(on - mx), axis=1, keepdims=True)
                       + jnp.exp(op - mx)) + mx
        return jnp.mean(logz - op)

    loss_s = ce(g2_ref[...], pos1_ref[:, 0:1])
    loss_t = ce(g1_ref[...], pos2_ref[:, 0:1])
    loss_ref[0, 0] = loss_s + loss_t


def _loss(g1, g2, pos1, pos2):
    return pl.pallas_call(
        _loss_body,
        out_shape=jax.ShapeDtypeStruct((1, 1), _F32),
        out_specs=pl.BlockSpec(memory_space=pltpu.SMEM),
    )(g1, g2, pos1, pos2)


def kernel(f_s, f_t, idx, contrast_idx, W_s, b_s, W_t, b_t,
           memory_v1, memory_v2):
    idx = idx.astype(jnp.int32)
    cidx = contrast_idx.astype(jnp.int32)
    v1, v2 = _embed(f_s, f_t, W_s, b_s, W_t, b_t)
    m1, m2 = _rowgather(idx, memory_v1, memory_v2)
    l1, l2, pos1, pos2 = _prep(idx, m1, m2, v1, v2)
    p1 = _mm_p(v2, memory_v1)
    g1 = _gather_bank(p1, cidx)
    p2, nm2 = _mm_bank(idx, v1, l2, memory_v2)
    g2 = _gather_bank(p2, cidx)
    nm1 = _copy_scatter_one(idx, l1, memory_v1, p2[:8, :128])
    loss = _loss(g1, g2, pos1, pos2)
    return loss[0, 0], nm1, nm2


# speedup vs baseline: 1.1068x; 1.1068x over previous
import functools

import jax
import jax.numpy as jnp
from jax import lax
from jax.experimental import pallas as pl
from jax.experimental.pallas import tpu as pltpu
from jax.experimental.pallas import tpu_sc as plsc

_D = 128
_B = 64
_K = 16384
_N = 100000
_T = 0.07
_M = 0.5
_RB = 8192
_CHUNK = 4096
_F32 = jnp.float32


def _embed_body(fs_ref, ft_ref, ws_ref, bs_ref, wt_ref, bt_ref, v1_ref, v2_ref):
    dn = (((1,), (1,)), ((), ()))
    x1 = lax.dot_general(fs_ref[...], ws_ref[...], dn,
                         preferred_element_type=_F32) + bs_ref[...]
    n1 = jnp.power(jnp.sum(x1 * x1, axis=1, keepdims=True), 0.5)
    v1_ref[...] = x1 / n1
    x2 = lax.dot_general(ft_ref[...], wt_ref[...], dn,
                         preferred_element_type=_F32) + bt_ref[...]
    n2 = jnp.power(jnp.sum(x2 * x2, axis=1, keepdims=True), 0.5)
    v2_ref[...] = x2 / n2


def _embed(f_s, f_t, W_s, b_s, W_t, b_t):
    return pl.pallas_call(
        _embed_body,
        out_shape=(jax.ShapeDtypeStruct((_B, _D), _F32),
                   jax.ShapeDtypeStruct((_B, _D), _F32)),
    )(f_s, f_t, W_s, b_s.reshape(1, _D), W_t, b_t.reshape(1, _D))


def _rowgather(idx, memory_v1, memory_v2):
    mesh = plsc.VectorSubcoreMesh(core_axis_name="c", subcore_axis_name="s")

    @functools.partial(
        pl.kernel,
        mesh=mesh,
        out_type=(jax.ShapeDtypeStruct((_B, _D), _F32),
                  jax.ShapeDtypeStruct((_B, _D), _F32)),
        scratch_types=[pltpu.VMEM((8,), jnp.int32),
                       pltpu.VMEM((8, _D), _F32),
                       pltpu.SemaphoreType.DMA],
    )
    def k(idx_hbm, mem1_hbm, mem2_hbm, m1_hbm, m2_hbm, idx_v, rows_v, sem):
        w = lax.axis_index("s") * 2 + lax.axis_index("c")

        @pl.when(w < 8)
        def _():
            pltpu.sync_copy(idx_hbm.at[pl.ds(w * 8, 8)], idx_v)
            pltpu.async_copy(mem1_hbm.at[idx_v], rows_v, sem).wait()
            pltpu.sync_copy(rows_v, m1_hbm.at[pl.ds(w * 8, 8)])
            pltpu.async_copy(mem2_hbm.at[idx_v], rows_v, sem).wait()
            pltpu.sync_copy(rows_v, m2_hbm.at[pl.ds(w * 8, 8)])

    return k(idx, memory_v1, memory_v2)


def _prep_body(idx_ref, m1_ref, m2_ref, v1_ref, v2_ref,
               l1_ref, l2_ref, pos1_ref, pos2_ref):
    ii = idx_ref[...]
    eq = (ii.T == ii)
    row = lax.broadcasted_iota(jnp.int32, (_B, _B), 0)
    col = lax.broadcasted_iota(jnp.int32, (_B, _B), 1)
    dup_after = jnp.any(eq & (col > row), axis=1, keepdims=True)
    win = (eq & jnp.logical_not(dup_after.T)).astype(_F32)

    m1 = m1_ref[...]
    m2 = m2_ref[...]
    v1 = v1_ref[...]
    v2 = v2_ref[...]
    l1 = m1 * _M + v1 * (1.0 - _M)
    l1 = l1 / jnp.power(jnp.sum(l1 * l1, axis=1, keepdims=True), 0.5)
    l2 = m2 * _M + v2 * (1.0 - _M)
    l2 = l2 / jnp.power(jnp.sum(l2 * l2, axis=1, keepdims=True), 0.5)
    l1_ref[...] = jnp.dot(win, l1, preferred_element_type=_F32)
    l2_ref[...] = jnp.dot(win, l2, preferred_element_type=_F32)
    pos1_ref[...] = jnp.broadcast_to(
        jnp.sum(m2 * v1, axis=1, keepdims=True), (_B, _D))
    pos2_ref[...] = jnp.broadcast_to(
        jnp.sum(m1 * v2, axis=1, keepdims=True), (_B, _D))


def _prep(idx, m1, m2, v1, v2):
    return pl.pallas_call(
        _prep_body,
        out_shape=tuple(jax.ShapeDtypeStruct((_B, _D), _F32) for _ in range(4)),
    )(idx.reshape(1, _B), m1, m2, v1, v2)


def _mm_bank_body(idx_ref, v_ref, l_ref, m_ref, p_ref, o_ref):
    dn = (((1,), (1,)), ((), ()))
    m = m_ref[...]
    p_ref[...] = lax.dot_general(v_ref[...], m, dn,
                                 preferred_element_type=_F32)
    o_ref[...] = m
    base = pl.program_id(0) * _RB
    for b in range(_B):
        j = idx_ref[b] - base

        @pl.when((j >= 0) & (j < _RB))
        def _():
            o_ref[pl.ds(j, 1), :] = l_ref[pl.ds(b, 1), :]


def _mm_p_body(v_ref, m_ref, p_ref):
    dn = (((1,), (1,)), ((), ()))
    p_ref[...] = lax.dot_general(v_ref[...], m_ref[...], dn,
                                 preferred_element_type=_F32)


def _mm_p(v, memory):
    grid = (pl.cdiv(_N, _RB),)
    full = pl.BlockSpec((_B, _D), lambda r: (0, 0))
    memb = pl.BlockSpec((_RB, _D), lambda r: (r, 0))
    pblk = pl.BlockSpec((_B, _RB), lambda r: (0, r))
    return pl.pallas_call(
        _mm_p_body,
        grid=grid,
        in_specs=[full, memb],
        out_specs=pblk,
        out_shape=jax.ShapeDtypeStruct((_B, _N), _F32),
        compiler_params=pltpu.CompilerParams(
            dimension_semantics=("arbitrary",)),
    )(v, memory)


def _copy_one_body(idx_ref, l_ref, m_ref, dep_ref, o_ref):
    o_ref[...] = m_ref[...]
    base = pl.program_id(0) * _RB
    for b in range(_B):
        j = idx_ref[b] - base

        @pl.when((j >= 0) & (j < _RB))
        def _():
            o_ref[pl.ds(j, 1), :] = l_ref[pl.ds(b, 1), :]


def _copy_scatter_one(idx, l, memory, dep):
    grid = (pl.cdiv(_N, _RB),)
    full = pl.BlockSpec((_B, _D), lambda r: (0, 0))
    memb = pl.BlockSpec((_RB, _D), lambda r: (r, 0))
    return pl.pallas_call(
        _copy_one_body,
        grid=grid,
        in_specs=[pl.BlockSpec(memory_space=pltpu.SMEM), full, memb,
                  pl.BlockSpec((8, 128), lambda r: (0, 0))],
        out_specs=memb,
        out_shape=jax.ShapeDtypeStruct((_N, _D), _F32),
        compiler_params=pltpu.CompilerParams(
            dimension_semantics=("arbitrary",)),
    )(idx, l, memory, dep)


def _mm_bank(idx, v, l, memory):
    grid = (pl.cdiv(_N, _RB),)
    full = pl.BlockSpec((_B, _D), lambda r: (0, 0))
    memb = pl.BlockSpec((_RB, _D), lambda r: (r, 0))
    pblk = pl.BlockSpec((_B, _RB), lambda r: (0, r))
    return pl.pallas_call(
        _mm_bank_body,
        grid=grid,
        in_specs=[pl.BlockSpec(memory_space=pltpu.SMEM), full, full, memb],
        out_specs=[pblk, memb],
        out_shape=(jax.ShapeDtypeStruct((_B, _N), _F32),
                   jax.ShapeDtypeStruct((_N, _D), _F32)),
        compiler_params=pltpu.CompilerParams(
            dimension_semantics=("arbitrary",)),
    )(idx, v, l, memory)


def _gather_bank(p, cidx):
    mesh = plsc.VectorSubcoreMesh(core_axis_name="c", subcore_axis_name="s")

    @functools.partial(
        pl.kernel,
        mesh=mesh,
        out_type=jax.ShapeDtypeStruct((_B, _K), _F32),
        scratch_types=[pltpu.VMEM((_N,), _F32),
                       pltpu.VMEM((2, _CHUNK), jnp.int32),
                       pltpu.VMEM((2, _CHUNK), _F32),
                       pltpu.SemaphoreType.DMA,
                       pltpu.SemaphoreType.DMA],
        compiler_params=pltpu.CompilerParams(needs_layout_passes=False),
    )
    def k(p_hbm, cidx_hbm, g_hbm,
          p_buf, idx_buf, out_buf, sem_i, sem_o):
        w = lax.axis_index("s") * 2 + lax.axis_index("c")

        nch = _K // _CHUNK

        def gather_row(b):
            icp = [pltpu.async_copy(cidx_hbm.at[b, pl.ds(0, _CHUNK)],
                                    idx_buf.at[0], sem_i)]
            pltpu.sync_copy(p_hbm.at[b], p_buf)
            ocp = [None] * nch
            for c in range(nch):
                s = c % 2
                icp[c].wait()
                if c + 1 < nch:
                    icp.append(pltpu.async_copy(
                        cidx_hbm.at[b, pl.ds((c + 1) * _CHUNK, _CHUNK)],
                        idx_buf.at[(c + 1) % 2], sem_i))
                if c >= 2:
                    ocp[c - 2].wait()

                def gl(j, carry):
                    for u in range(4):
                        o = j * 64 + u * 16
                        vi = idx_buf[s, pl.ds(o, 16)]
                        out_buf[s, pl.ds(o, 16)] = \
                            plsc.load_gather(p_buf, [vi])
                    return carry

                lax.fori_loop(0, _CHUNK // 64, gl, 0)
                ocp[c] = pltpu.async_copy(
                    out_buf.at[s], g_hbm.at[b, pl.ds(c * _CHUNK, _CHUNK)],
                    sem_o)
            ocp[nch - 2].wait()
            ocp[nch - 1].wait()

        gather_row(w * 2)
        gather_row(w * 2 + 1)

    return k(p, cidx)


def _ce_body(g_ref, pos_ref, loss_ref):
    inv_t = 1.0 / _T
    scale = float(_N) / float(_B * (_K + 1))
    en = jnp.exp(g_ref[...] * inv_t)
    ep = jnp.exp(pos_ref[:, 0:1] * inv_t)
    z = jnp.sum(jnp.sum(en, axis=1, keepdims=True) + ep) * scale
    on = en / z
    op = ep / z
    mx = jnp.maximum(jnp.max(on, axis=1, keepdims=True), op)
    logz = jnp.log(jnp.sum(jnp.exp(on - mx), axis=1, keepdims=True)
                   + jnp.exp(op - mx)) + mx
    loss_ref[0, 0] = jnp.mean(logz - op)


def _ce(g, pos):
    return pl.pallas_call(
        _ce_body,
        out_shape=jax.ShapeDtypeStruct((1, 1), _F32),
        out_specs=pl.BlockSpec(memory_space=pltpu.SMEM),
    )(g, pos)


def kernel(f_s, f_t, idx, contrast_idx, W_s, b_s, W_t, b_t,
           memory_v1, memory_v2):
    idx = idx.astype(jnp.int32)
    cidx = contrast_idx.astype(jnp.int32)
    v1, v2 = _embed(f_s, f_t, W_s, b_s, W_t, b_t)
    m1, m2 = _rowgather(idx, memory_v1, memory_v2)
    l1, l2, pos1, pos2 = _prep(idx, m1, m2, v1, v2)
    p1, nm1 = _mm_bank(idx, v2, l1, memory_v1)
    g1 = _gather_bank(p1, cidx)
    p2, nm2 = _mm_bank(idx, v1, l2, memory_v2)
    g2 = _gather_bank(p2, cidx)
    loss_t = _ce(g1, pos2)
    loss_s = _ce(g2, pos1)
    return loss_s[0, 0] + loss_t[0, 0], nm1, nm2


# speedup vs baseline: 1.1090x; 1.0020x over previous
import functools

import jax
import jax.numpy as jnp
from jax import lax
from jax.experimental import pallas as pl
from jax.experimental.pallas import tpu as pltpu
from jax.experimental.pallas import tpu_sc as plsc

_D = 128
_B = 64
_K = 16384
_N = 100000
_T = 0.07
_M = 0.5
_RB = 16384
_CHUNK = 4096
_F32 = jnp.float32


def _embed_body(fs_ref, ft_ref, ws_ref, bs_ref, wt_ref, bt_ref, v1_ref, v2_ref):
    dn = (((1,), (1,)), ((), ()))
    x1 = lax.dot_general(fs_ref[...], ws_ref[...], dn,
                         preferred_element_type=_F32) + bs_ref[...]
    n1 = jnp.power(jnp.sum(x1 * x1, axis=1, keepdims=True), 0.5)
    v1_ref[...] = x1 / n1
    x2 = lax.dot_general(ft_ref[...], wt_ref[...], dn,
                         preferred_element_type=_F32) + bt_ref[...]
    n2 = jnp.power(jnp.sum(x2 * x2, axis=1, keepdims=True), 0.5)
    v2_ref[...] = x2 / n2


def _embed(f_s, f_t, W_s, b_s, W_t, b_t):
    return pl.pallas_call(
        _embed_body,
        out_shape=(jax.ShapeDtypeStruct((_B, _D), _F32),
                   jax.ShapeDtypeStruct((_B, _D), _F32)),
    )(f_s, f_t, W_s, b_s.reshape(1, _D), W_t, b_t.reshape(1, _D))


def _rowgather(idx, memory_v1, memory_v2):
    mesh = plsc.VectorSubcoreMesh(core_axis_name="c", subcore_axis_name="s")

    @functools.partial(
        pl.kernel,
        mesh=mesh,
        out_type=(jax.ShapeDtypeStruct((_B, _D), _F32),
                  jax.ShapeDtypeStruct((_B, _D), _F32)),
        scratch_types=[pltpu.VMEM((8,), jnp.int32),
                       pltpu.VMEM((8, _D), _F32),
                       pltpu.SemaphoreType.DMA],
    )
    def k(idx_hbm, mem1_hbm, mem2_hbm, m1_hbm, m2_hbm, idx_v, rows_v, sem):
        w = lax.axis_index("s") * 2 + lax.axis_index("c")

        @pl.when(w < 8)
        def _():
            pltpu.sync_copy(idx_hbm.at[pl.ds(w * 8, 8)], idx_v)
            pltpu.async_copy(mem1_hbm.at[idx_v], rows_v, sem).wait()
            pltpu.sync_copy(rows_v, m1_hbm.at[pl.ds(w * 8, 8)])
            pltpu.async_copy(mem2_hbm.at[idx_v], rows_v, sem).wait()
            pltpu.sync_copy(rows_v, m2_hbm.at[pl.ds(w * 8, 8)])

    return k(idx, memory_v1, memory_v2)


def _prep_body(idx_ref, m1_ref, m2_ref, v1_ref, v2_ref,
               l1_ref, l2_ref, pos1_ref, pos2_ref):
    ii = idx_ref[...]
    eq = (ii.T == ii)
    row = lax.broadcasted_iota(jnp.int32, (_B, _B), 0)
    col = lax.broadcasted_iota(jnp.int32, (_B, _B), 1)
    dup_after = jnp.any(eq & (col > row), axis=1, keepdims=True)
    win = (eq & jnp.logical_not(dup_after.T)).astype(_F32)

    m1 = m1_ref[...]
    m2 = m2_ref[...]
    v1 = v1_ref[...]
    v2 = v2_ref[...]
    l1 = m1 * _M + v1 * (1.0 - _M)
    l1 = l1 / jnp.power(jnp.sum(l1 * l1, axis=1, keepdims=True), 0.5)
    l2 = m2 * _M + v2 * (1.0 - _M)
    l2 = l2 / jnp.power(jnp.sum(l2 * l2, axis=1, keepdims=True), 0.5)
    l1_ref[...] = jnp.dot(win, l1, preferred_element_type=_F32)
    l2_ref[...] = jnp.dot(win, l2, preferred_element_type=_F32)
    pos1_ref[...] = jnp.broadcast_to(
        jnp.sum(m2 * v1, axis=1, keepdims=True), (_B, _D))
    pos2_ref[...] = jnp.broadcast_to(
        jnp.sum(m1 * v2, axis=1, keepdims=True), (_B, _D))


def _prep(idx, m1, m2, v1, v2):
    return pl.pallas_call(
        _prep_body,
        out_shape=tuple(jax.ShapeDtypeStruct((_B, _D), _F32) for _ in range(4)),
    )(idx.reshape(1, _B), m1, m2, v1, v2)


def _mm_bank_body(idx_ref, v_ref, l_ref, m_ref, p_ref, o_ref):
    dn = (((1,), (1,)), ((), ()))
    m = m_ref[...]
    p_ref[...] = lax.dot_general(v_ref[...], m, dn,
                                 preferred_element_type=_F32)
    o_ref[...] = m
    base = pl.program_id(0) * _RB
    for b in range(_B):
        j = idx_ref[b] - base

        @pl.when((j >= 0) & (j < _RB))
        def _():
            o_ref[pl.ds(j, 1), :] = l_ref[pl.ds(b, 1), :]


def _mm_p_body(v_ref, m_ref, p_ref):
    dn = (((1,), (1,)), ((), ()))
    p_ref[...] = lax.dot_general(v_ref[...], m_ref[...], dn,
                                 preferred_element_type=_F32)


def _mm_p(v, memory):
    grid = (pl.cdiv(_N, _RB),)
    full = pl.BlockSpec((_B, _D), lambda r: (0, 0))
    memb = pl.BlockSpec((_RB, _D), lambda r: (r, 0))
    pblk = pl.BlockSpec((_B, _RB), lambda r: (0, r))
    return pl.pallas_call(
        _mm_p_body,
        grid=grid,
        in_specs=[full, memb],
        out_specs=pblk,
        out_shape=jax.ShapeDtypeStruct((_B, _N), _F32),
        compiler_params=pltpu.CompilerParams(
            dimension_semantics=("arbitrary",)),
    )(v, memory)


def _copy_one_body(idx_ref, l_ref, m_ref, dep_ref, o_ref):
    o_ref[...] = m_ref[...]
    base = pl.program_id(0) * _RB
    for b in range(_B):
        j = idx_ref[b] - base

        @pl.when((j >= 0) & (j < _RB))
        def _():
            o_ref[pl.ds(j, 1), :] = l_ref[pl.ds(b, 1), :]


def _copy_scatter_one(idx, l, memory, dep):
    grid = (pl.cdiv(_N, _RB),)
    full = pl.BlockSpec((_B, _D), lambda r: (0, 0))
    memb = pl.BlockSpec((_RB, _D), lambda r: (r, 0))
    return pl.pallas_call(
        _copy_one_body,
        grid=grid,
        in_specs=[pl.BlockSpec(memory_space=pltpu.SMEM), full, memb,
                  pl.BlockSpec((8, 128), lambda r: (0, 0))],
        out_specs=memb,
        out_shape=jax.ShapeDtypeStruct((_N, _D), _F32),
        compiler_params=pltpu.CompilerParams(
            dimension_semantics=("arbitrary",)),
    )(idx, l, memory, dep)


def _mm_bank(idx, v, l, memory):
    grid = (pl.cdiv(_N, _RB),)
    full = pl.BlockSpec((_B, _D), lambda r: (0, 0))
    memb = pl.BlockSpec((_RB, _D), lambda r: (r, 0))
    pblk = pl.BlockSpec((_B, _RB), lambda r: (0, r))
    return pl.pallas_call(
        _mm_bank_body,
        grid=grid,
        in_specs=[pl.BlockSpec(memory_space=pltpu.SMEM), full, full, memb],
        out_specs=[pblk, memb],
        out_shape=(jax.ShapeDtypeStruct((_B, _N), _F32),
                   jax.ShapeDtypeStruct((_N, _D), _F32)),
        compiler_params=pltpu.CompilerParams(
            dimension_semantics=("arbitrary",)),
    )(idx, v, l, memory)


def _gather_bank(p, cidx):
    mesh = plsc.VectorSubcoreMesh(core_axis_name="c", subcore_axis_name="s")

    @functools.partial(
        pl.kernel,
        mesh=mesh,
        out_type=jax.ShapeDtypeStruct((_B, _K), _F32),
        scratch_types=[pltpu.VMEM((_N,), _F32),
                       pltpu.VMEM((2, _CHUNK), jnp.int32),
                       pltpu.VMEM((2, _CHUNK), _F32),
                       pltpu.SemaphoreType.DMA,
                       pltpu.SemaphoreType.DMA],
        compiler_params=pltpu.CompilerParams(needs_layout_passes=False),
    )
    def k(p_hbm, cidx_hbm, g_hbm,
          p_buf, idx_buf, out_buf, sem_i, sem_o):
        w = lax.axis_index("s") * 2 + lax.axis_index("c")

        nch = _K // _CHUNK

        def gather_row(b):
            icp = [pltpu.async_copy(cidx_hbm.at[b, pl.ds(0, _CHUNK)],
                                    idx_buf.at[0], sem_i)]
            pltpu.sync_copy(p_hbm.at[b], p_buf)
            ocp = [None] * nch
            for c in range(nch):
                s = c % 2
                icp[c].wait()
                if c + 1 < nch:
                    icp.append(pltpu.async_copy(
                        cidx_hbm.at[b, pl.ds((c + 1) * _CHUNK, _CHUNK)],
                        idx_buf.at[(c + 1) % 2], sem_i))
                if c >= 2:
                    ocp[c - 2].wait()

                def gl(j, carry):
                    for u in range(4):
                        o = j * 64 + u * 16
                        vi = idx_buf[s, pl.ds(o, 16)]
                        out_buf[s, pl.ds(o, 16)] = \
                            plsc.load_gather(p_buf, [vi])
                    return carry

                lax.fori_loop(0, _CHUNK // 64, gl, 0)
                ocp[c] = pltpu.async_copy(
                    out_buf.at[s], g_hbm.at[b, pl.ds(c * _CHUNK, _CHUNK)],
                    sem_o)
            ocp[nch - 2].wait()
            ocp[nch - 1].wait()

        gather_row(w * 2)
        gather_row(w * 2 + 1)

    return k(p, cidx)


def _ce_body(g_ref, pos_ref, loss_ref):
    inv_t = 1.0 / _T
    scale = float(_N) / float(_B * (_K + 1))
    en = jnp.exp(g_ref[...] * inv_t)
    ep = jnp.exp(pos_ref[:, 0:1] * inv_t)
    z = jnp.sum(jnp.sum(en, axis=1, keepdims=True) + ep) * scale
    on = en / z
    op = ep / z
    mx = jnp.maximum(jnp.max(on, axis=1, keepdims=True), op)
    logz = jnp.log(jnp.sum(jnp.exp(on - mx), axis=1, keepdims=True)
                   + jnp.exp(op - mx)) + mx
    loss_ref[0, 0] = jnp.mean(logz - op)


def _ce(g, pos):
    return pl.pallas_call(
        _ce_body,
        out_shape=jax.ShapeDtypeStruct((1, 1), _F32),
        out_specs=pl.BlockSpec(memory_space=pltpu.SMEM),
    )(g, pos)


def kernel(f_s, f_t, idx, contrast_idx, W_s, b_s, W_t, b_t,
           memory_v1, memory_v2):
    idx = idx.astype(jnp.int32)
    cidx = contrast_idx.astype(jnp.int32)
    v1, v2 = _embed(f_s, f_t, W_s, b_s, W_t, b_t)
    m1, m2 = _rowgather(idx, memory_v1, memory_v2)
    l1, l2, pos1, pos2 = _prep(idx, m1, m2, v1, v2)
    p1, nm1 = _mm_bank(idx, v2, l1, memory_v1)
    g1 = _gather_bank(p1, cidx)
    p2, nm2 = _mm_bank(idx, v1, l2, memory_v2)
    g2 = _gather_bank(p2, cidx)
    loss_t = _ce(g1, pos2)
    loss_s = _ce(g2, pos1)
    return loss_s[0, 0] + loss_t[0, 0], nm1, nm2
